# baseline (speedup 1.0000x reference)
import functools

import jax
import jax.numpy as jnp
from jax import lax
from jax.experimental import pallas as pl
from jax.experimental.pallas import tpu as pltpu
from jax.experimental.pallas import tpu_sc as plsc

VOCAB = 100000
HIDDEN = 512
EMBED = 128

M_BLK = 2000

NC = 2
NS = 16
NW = NC * NS


def _matmul_body(base_ref, w_ref, out_ref):
    out_ref[...] = jnp.dot(base_ref[...], w_ref[...],
                           preferred_element_type=jnp.float32)


def _compose_table(base, W):
    grid = VOCAB // M_BLK
    return pl.pallas_call(
        _matmul_body,
        grid=(grid,),
        in_specs=[
            pl.BlockSpec((M_BLK, HIDDEN), lambda i: (i, 0)),
            pl.BlockSpec((HIDDEN, EMBED), lambda i: (0, 0)),
        ],
        out_specs=pl.BlockSpec((M_BLK, EMBED), lambda i: (i, 0)),
        out_shape=jax.ShapeDtypeStruct((VOCAB, EMBED), jnp.float32),
    )(base, W)


def _make_gather(bsz, seq):
    seq_per_w = bsz // NW
    mesh = plsc.VectorSubcoreMesh(core_axis_name="c", subcore_axis_name="s")

    @functools.partial(
        pl.kernel,
        mesh=mesh,
        out_type=jax.ShapeDtypeStruct((bsz, seq, EMBED), jnp.float32),
        scratch_types=[
            pltpu.VMEM((seq_per_w, seq), jnp.int32),
            pltpu.VMEM((seq, EMBED), jnp.float32),
            pltpu.VMEM((seq, EMBED), jnp.float32),
            pltpu.SemaphoreType.DMA,
            pltpu.SemaphoreType.DMA,
            pltpu.SemaphoreType.DMA,
        ],
    )
    def gather_k(table_hbm, idx_hbm, out_hbm, idx_v, buf0, buf1, gsem0, gsem1,
                 osem):
        wid = lax.axis_index("s") * NC + lax.axis_index("c")
        seq0 = wid * seq_per_w
        pltpu.sync_copy(idx_hbm.at[wid], idx_v)

        bufs = (buf0, buf1)
        gsems = (gsem0, gsem1)

        pltpu.async_copy(table_hbm.at[idx_v.at[0]], buf0, gsem0)

        def body(g, _):
            slot = lax.rem(g, 2)

            @pl.when(g + 1 < seq_per_w)
            def _():
                nxt = lax.rem(g + 1, 2)
                for b in range(2):
                    @pl.when(nxt == b)
                    def _():
                        pltpu.async_copy(table_hbm.at[idx_v.at[g + 1]],
                                         bufs[b], gsems[b])

            for b in range(2):
                @pl.when(slot == b)
                def _():
                    pltpu.make_async_copy(table_hbm.at[idx_v.at[g]],
                                          bufs[b], gsems[b]).wait()
                    pltpu.async_copy(bufs[b], out_hbm.at[seq0 + g], osem)
                    pltpu.make_async_copy(bufs[b], out_hbm.at[seq0 + g],
                                          osem).wait()
            return 0

        lax.fori_loop(0, seq_per_w, body, 0)

    return gather_k


def kernel(data, base, W):
    d = jnp.squeeze(data, axis=2)
    bsz, seq = d.shape
    idx = d.astype(jnp.int32).reshape(NW, bsz // NW, seq)

    var = _compose_table(base, W)
    out = _make_gather(bsz, seq)(var, idx)
    return out


# speedup vs baseline: 1.1020x; 1.1020x over previous
import functools

import jax
import jax.numpy as jnp
from jax import lax
from jax.experimental import pallas as pl
from jax.experimental.pallas import tpu as pltpu
from jax.experimental.pallas import tpu_sc as plsc

VOCAB = 100000
HIDDEN = 512
EMBED = 128

M_BLK = 2000

NC = 2
NS = 16
NW = NC * NS
SEQ_PER_CHUNK = 8


def _matmul_body(base_ref, w_ref, out_ref):
    out_ref[...] = jnp.dot(base_ref[...], w_ref[...],
                           preferred_element_type=jnp.float32)


def _compose_table(base, W):
    grid = VOCAB // M_BLK
    return pl.pallas_call(
        _matmul_body,
        grid=(grid,),
        in_specs=[
            pl.BlockSpec((M_BLK, HIDDEN), lambda i: (i, 0)),
            pl.BlockSpec((HIDDEN, EMBED), lambda i: (0, 0)),
        ],
        out_specs=pl.BlockSpec((M_BLK, EMBED), lambda i: (i, 0)),
        out_shape=jax.ShapeDtypeStruct((VOCAB, EMBED), jnp.float32),
    )(base, W)


def _make_gather(bsz, seq):
    seq_per_w = bsz // NW
    n_chunks = seq_per_w // SEQ_PER_CHUNK
    n_g = SEQ_PER_CHUNK
    mesh = plsc.VectorSubcoreMesh(core_axis_name="c", subcore_axis_name="s")

    @functools.partial(
        pl.kernel,
        mesh=mesh,
        out_type=jax.ShapeDtypeStruct((bsz, seq, EMBED), jnp.float32),
        scratch_types=[
            pltpu.VMEM((n_chunks * n_g, seq), jnp.int32),
            pltpu.VMEM((SEQ_PER_CHUNK, seq, EMBED), jnp.float32),
            pltpu.VMEM((SEQ_PER_CHUNK, seq, EMBED), jnp.float32),
            pltpu.SemaphoreType.DMA,
            pltpu.SemaphoreType.DMA,
            pltpu.SemaphoreType.DMA,
        ],
    )
    def gather_k(table_hbm, idx_hbm, out_hbm, idx_v, buf0, buf1, gsem0, gsem1,
                 osem):
        wid = lax.axis_index("s") * NC + lax.axis_index("c")
        seq0 = wid * seq_per_w
        pltpu.sync_copy(idx_hbm.at[wid], idx_v)

        bufs = (buf0, buf1)
        gsems = (gsem0, gsem1)

        def fire(c, b):
            for j in range(n_g):
                pltpu.async_copy(table_hbm.at[idx_v.at[c * n_g + j]],
                                 bufs[b].at[j], gsems[b])

        def drain(c, b):
            for j in range(n_g):
                pltpu.make_async_copy(table_hbm.at[idx_v.at[c * n_g + j]],
                                      bufs[b].at[j], gsems[b]).wait()

        fire(0, 0)

        def body(g, _):
            slot = lax.rem(g, 2)

            @pl.when(g + 1 < n_chunks)
            def _():
                nxt = lax.rem(g + 1, 2)
                for b in range(2):
                    @pl.when(nxt == b)
                    def _():
                        fire(g + 1, b)

            for b in range(2):
                @pl.when(slot == b)
                def _():
                    drain(g, b)
                    dst = out_hbm.at[pl.ds(seq0 + g * SEQ_PER_CHUNK,
                                           SEQ_PER_CHUNK)]
                    pltpu.async_copy(bufs[b], dst, osem)
                    pltpu.make_async_copy(bufs[b], dst, osem).wait()
            return 0

        lax.fori_loop(0, n_chunks, body, 0)

    return gather_k


def kernel(data, base, W):
    d = jnp.squeeze(data, axis=2)
    bsz, seq = d.shape
    idx = d.astype(jnp.int32).reshape(NW, bsz // NW, seq)

    var = _compose_table(base, W)
    out = _make_gather(bsz, seq)(var, idx)
    return out


# speedup vs baseline: 1.1022x; 1.0002x over previous
import functools

import jax
import jax.numpy as jnp
from jax import lax
from jax.experimental import pallas as pl
from jax.experimental.pallas import tpu as pltpu
from jax.experimental.pallas import tpu_sc as plsc

VOCAB = 100000
HIDDEN = 512
EMBED = 128

M_BLK = 2000

NC = 2
NS = 16
NW = NC * NS
SEQ_PER_CHUNK = 4
NBUF = 4


def _matmul_body(base_ref, w_ref, out_ref):
    out_ref[...] = jnp.dot(base_ref[...], w_ref[...],
                           preferred_element_type=jnp.float32)


def _compose_table(base, W):
    grid = VOCAB // M_BLK
    return pl.pallas_call(
        _matmul_body,
        grid=(grid,),
        in_specs=[
            pl.BlockSpec((M_BLK, HIDDEN), lambda i: (i, 0)),
            pl.BlockSpec((HIDDEN, EMBED), lambda i: (0, 0)),
        ],
        out_specs=pl.BlockSpec((M_BLK, EMBED), lambda i: (i, 0)),
        out_shape=jax.ShapeDtypeStruct((VOCAB, EMBED), jnp.float32),
    )(base, W)


def _make_gather(bsz, seq):
    seq_per_w = bsz // NW
    n_chunks = seq_per_w // SEQ_PER_CHUNK
    n_g = SEQ_PER_CHUNK
    mesh = plsc.VectorSubcoreMesh(core_axis_name="c", subcore_axis_name="s")

    @functools.partial(
        pl.kernel,
        mesh=mesh,
        out_type=jax.ShapeDtypeStruct((bsz, seq, EMBED), jnp.float32),
        scratch_types=[
            pltpu.VMEM((n_chunks, n_g, seq), jnp.int32),
        ] + [pltpu.VMEM((SEQ_PER_CHUNK, seq, EMBED), jnp.float32)
             for _ in range(NBUF)]
        + [pltpu.SemaphoreType.DMA for _ in range(2 * NBUF)],
    )
    def gather_k(table_hbm, idx_hbm, out_hbm, idx_v, *bufs_sems):
        bufs = bufs_sems[:NBUF]
        gsems = bufs_sems[NBUF:2 * NBUF]
        osems = bufs_sems[2 * NBUF:]

        wid = lax.axis_index("s") * NC + lax.axis_index("c")
        seq0 = wid * seq_per_w
        pltpu.sync_copy(idx_hbm.at[wid], idx_v)

        def out_at(c):
            return out_hbm.at[pl.ds(seq0 + c * SEQ_PER_CHUNK, SEQ_PER_CHUNK)]

        def fire(c, b):
            for j in range(n_g):
                pltpu.async_copy(table_hbm.at[idx_v.at[c, j]],
                                 bufs[b].at[j], gsems[b])

        def drain(c, b):
            pltpu.make_async_copy(out_at(c), bufs[b], gsems[b]).wait()

        def fire_write(c, b):
            pltpu.async_copy(bufs[b], out_at(c), osems[b])

        def wait_write(c, b):
            pltpu.make_async_copy(bufs[b], out_at(c), osems[b]).wait()

        def on_slot(slot, fn):
            for b in range(NBUF):
                @pl.when(slot == b)
                def _():
                    fn(b)

        for c in range(NBUF - 1):
            fire(c, c)

        def body(g, _):
            slot = lax.rem(g, NBUF)
            on_slot(slot, lambda b: drain(g, b))
            on_slot(slot, lambda b: fire_write(g, b))

            @pl.when(g + NBUF - 1 < n_chunks)
            def _():
                nslot = lax.rem(g + NBUF - 1, NBUF)

                @pl.when(g >= 1)
                def _():
                    on_slot(nslot, lambda b: wait_write(g - 1, b))
                on_slot(nslot, lambda b: fire(g + NBUF - 1, b))
            return 0

        lax.fori_loop(0, n_chunks, body, 0)

        for c in range(n_chunks - NBUF, n_chunks):
            wait_write(c, c % NBUF)

    return gather_k


def kernel(data, base, W):
    d = jnp.squeeze(data, axis=2)
    bsz, seq = d.shape
    idx = d.astype(jnp.int32).reshape(
        NW, (bsz // NW) // SEQ_PER_CHUNK, SEQ_PER_CHUNK, seq)

    var = _compose_table(base, W)
    out = _make_gather(bsz, seq)(var, idx)
    return out
